# baseline (speedup 1.0000x reference)
import jax
import jax.numpy as jnp
from jax import lax
from jax.experimental import pallas as pl
from jax.experimental.pallas import tpu as pltpu
from jax.experimental.pallas import tpu_sc as plsc

B, S, D, H, O, E, K = 1, 2048, 1024, 1024, 1024, 8, 2
TB = 256
NSLOT = S * K // TB + E - 1
NPAD = NSLOT * TB
SUB = 8
NC, NS = 2, 16
NW = NC * NS
SPW = S * SUB // NW
IPW = S * SUB // NW // 128


def _dot(a, b, precision=None):
    return jax.lax.dot_general(
        a, b, (((1,), (0,)), ((), ())),
        precision=precision, preferred_element_type=jnp.float32)


def _a1_body(x_ref, nw_ref, rwt_ref, ltri_ref, sh1_ref, sh1b_ref, sh2_ref,
             sh2b_ref, nout_ref, sh_ref, d1s_ref, d2s_ref,
             w1o_ref, w2o_ref, ex_ref):
    x = x_ref[...]
    var = jnp.mean(x * x, axis=-1, keepdims=True)
    normed = x * jax.lax.rsqrt(var + 1e-8) * nw_ref[...]
    nout_ref[...] = normed

    logits = _dot(normed, rwt_ref[...])
    probs = 1.0 / (1.0 + jnp.exp(-logits))
    eidx = jax.lax.broadcasted_iota(jnp.int32, probs.shape, 1)
    m1 = jnp.max(probs, axis=-1, keepdims=True)
    i1 = jnp.min(jnp.where(probs == m1, eidx, E), axis=-1, keepdims=True)
    probs2 = jnp.where(eidx == i1, -1.0, probs)
    m2 = jnp.max(probs2, axis=-1, keepdims=True)
    i2 = jnp.min(jnp.where(probs2 == m2, eidx, E), axis=-1, keepdims=True)
    denom = m1 + m2 + 1e-6
    w1o_ref[...] = m1 / denom
    w2o_ref[...] = m2 / denom

    oh1 = (eidx == i1).astype(jnp.float32)
    oh2 = (eidx == i2).astype(jnp.float32)
    sel = oh1 + oh2
    prefix = _dot(ltri_ref[...], sel.astype(jnp.bfloat16))
    g = prefix[S - 1:S, :] + sel[S - 1:S, :]
    scnt = jnp.ceil(g * (1.0 / TB))
    u_r = jax.lax.broadcasted_iota(jnp.int32, (E, E), 0)
    u_c = jax.lax.broadcasted_iota(jnp.int32, (E, E), 1)
    ustrict = (u_r < u_c).astype(jnp.float32)
    spoff = _dot(scnt, ustrict)
    poff = spoff * float(TB)
    dest1 = jnp.sum(oh1 * (poff + prefix), axis=-1, keepdims=True)
    dest2 = jnp.sum(oh2 * (poff + prefix), axis=-1, keepdims=True)
    j8 = jax.lax.broadcasted_iota(jnp.int32, (1, SUB), 1).astype(jnp.float32)
    d1s_ref[...] = (dest1 * float(SUB) + j8).astype(jnp.int32)
    d2s_ref[...] = (dest2 * float(SUB) + j8).astype(jnp.int32)
    s_i = jax.lax.broadcasted_iota(jnp.int32, (32, 1), 0).astype(jnp.float32)
    cnt = jnp.sum((jnp.broadcast_to(spoff, (32, E)) <= s_i).astype(
        jnp.float32), axis=-1, keepdims=True)
    ex_ref[...] = (cnt - 1.0).astype(jnp.int32)

    for t in range(4):
        nb = normed[t * 512:(t + 1) * 512].astype(jnp.bfloat16)
        h = jnp.maximum(_dot(nb, sh1_ref[...]) + sh1b_ref[...], 0.0)
        sh_ref[pl.ds(t * 512, 512), :] = (
            _dot(h.astype(jnp.bfloat16), sh2_ref[...]) + sh2b_ref[...])


def _run_a1(x, norm_w, rwt, ltri, sh1t, sh1b, sh2t, sh2b):
    c = lambda i: (0, 0)
    return pl.pallas_call(
        _a1_body,
        grid=(1,),
        in_specs=[pl.BlockSpec((S, D), c),
                  pl.BlockSpec((1, D), c),
                  pl.BlockSpec((D, E), c),
                  pl.BlockSpec((S, S), c),
                  pl.BlockSpec((D, H), c),
                  pl.BlockSpec((1, H), c),
                  pl.BlockSpec((H, O), c),
                  pl.BlockSpec((1, O), c)],
        out_specs=[pl.BlockSpec((S, D), c),
                   pl.BlockSpec((S, O), c),
                   pl.BlockSpec((S, SUB), c),
                   pl.BlockSpec((S, SUB), c),
                   pl.BlockSpec((S, 1), c),
                   pl.BlockSpec((S, 1), c),
                   pl.BlockSpec((32, 1), c)],
        out_shape=[jax.ShapeDtypeStruct((S, D), jnp.float32),
                   jax.ShapeDtypeStruct((S, O), jnp.float32),
                   jax.ShapeDtypeStruct((S, SUB), jnp.int32),
                   jax.ShapeDtypeStruct((S, SUB), jnp.int32),
                   jax.ShapeDtypeStruct((S, 1), jnp.float32),
                   jax.ShapeDtypeStruct((S, 1), jnp.float32),
                   jax.ShapeDtypeStruct((32, 1), jnp.int32)],
        compiler_params=pltpu.CompilerParams(
            dimension_semantics=("arbitrary",)),
    )(x, norm_w.reshape(1, D), rwt, ltri, sh1t, sh1b.reshape(1, H),
      sh2t, sh2b.reshape(1, O))


def _b_body(xp_hbm, d1_hbm, d2_hbm, out_hbm, idx_v, rows_v, sem):
    wid = lax.axis_index("s") * NC + lax.axis_index("c")
    base = wid * SPW
    pltpu.sync_copy(xp_hbm.at[pl.ds(base, SPW)], rows_v)
    for dm in (d1_hbm, d2_hbm):
        for j in range(IPW):
            pltpu.sync_copy(dm.at[wid * IPW + j], idx_v)
            pltpu.async_copy(rows_v.at[pl.ds(j * 128, 128)],
                             out_hbm.at[idx_v], sem).wait()


def _run_b(xp_lin, d1m, d2m):
    mesh = plsc.VectorSubcoreMesh(
        core_axis_name="c", subcore_axis_name="s",
        num_cores=NC, num_subcores=NS)
    return pl.kernel(
        _b_body,
        out_type=jax.ShapeDtypeStruct((NPAD * SUB, 128), jnp.float32),
        mesh=mesh,
        scratch_types=[pltpu.VMEM((128,), jnp.int32),
                       pltpu.VMEM((SPW, 128), jnp.float32),
                       pltpu.SemaphoreType.DMA],
    )(xp_lin, d1m, d2m)


def _c_body(ex_ref, x_ref, w1_ref, b1_ref, w2_ref, b2_ref, o_ref):
    xb = jnp.reshape(x_ref[...], (TB, D)).astype(jnp.bfloat16)
    h = jnp.maximum(_dot(xb, w1_ref[0]) + b1_ref[0], 0.0)
    y = _dot(h.astype(jnp.bfloat16), w2_ref[0]) + b2_ref[0]
    o_ref[...] = jnp.reshape(y, (TB * SUB, 128))


def _run_c(ex_tab, x_padded_lin, w1t, b1e, w2t, b2e):
    return pl.pallas_call(
        _c_body,
        grid_spec=pltpu.PrefetchScalarGridSpec(
            num_scalar_prefetch=1,
            grid=(NSLOT,),
            in_specs=[
                pl.BlockSpec((TB * SUB, 128), lambda s, ex: (s, 0)),
                pl.BlockSpec((1, D, H), lambda s, ex: (ex[s], 0, 0)),
                pl.BlockSpec((1, 1, H), lambda s, ex: (ex[s], 0, 0)),
                pl.BlockSpec((1, H, O), lambda s, ex: (ex[s], 0, 0)),
                pl.BlockSpec((1, 1, O), lambda s, ex: (ex[s], 0, 0)),
            ],
            out_specs=pl.BlockSpec((TB * SUB, 128), lambda s, ex: (s, 0)),
        ),
        out_shape=jax.ShapeDtypeStruct((NPAD * SUB, 128), jnp.float32),
        compiler_params=pltpu.CompilerParams(
            dimension_semantics=("arbitrary",)),
    )(ex_tab, x_padded_lin, w1t, b1e.reshape(E, 1, H), w2t,
      b2e.reshape(E, 1, O))


def _d_body(yp_hbm, d1_hbm, d2_hbm, r1_hbm, r2_hbm, idx_v, rows_v, sem):
    wid = lax.axis_index("s") * NC + lax.axis_index("c")
    base = wid * SPW
    for dm, r_hbm in ((d1_hbm, r1_hbm), (d2_hbm, r2_hbm)):
        for j in range(IPW):
            pltpu.sync_copy(dm.at[wid * IPW + j], idx_v)
            pltpu.async_copy(yp_hbm.at[idx_v],
                             rows_v.at[pl.ds(j * 128, 128)], sem).wait()
        pltpu.sync_copy(rows_v, r_hbm.at[pl.ds(base, SPW)])


def _run_d(yp_lin, d1m, d2m):
    mesh = plsc.VectorSubcoreMesh(
        core_axis_name="c", subcore_axis_name="s",
        num_cores=NC, num_subcores=NS)
    return pl.kernel(
        _d_body,
        out_type=[jax.ShapeDtypeStruct((S * SUB, 128), jnp.float32),
                  jax.ShapeDtypeStruct((S * SUB, 128), jnp.float32)],
        mesh=mesh,
        scratch_types=[pltpu.VMEM((128,), jnp.int32),
                       pltpu.VMEM((SPW, 128), jnp.float32),
                       pltpu.SemaphoreType.DMA],
    )(yp_lin, d1m, d2m)


def _e_body(sh_ref, r1_ref, r2_ref, w1_ref, w2_ref, o_ref):
    o_ref[...] = (sh_ref[...] + w1_ref[...] * r1_ref[...]
                  + w2_ref[...] * r2_ref[...])


def _run_e(shared_lin, r1_lin, r2_lin, w1s, w2s):
    m = lambda i: (i, 0)
    blk = 2048
    return pl.pallas_call(
        _e_body,
        grid=(S * SUB // blk,),
        in_specs=[pl.BlockSpec((blk, 128), m),
                  pl.BlockSpec((blk, 128), m),
                  pl.BlockSpec((blk, 128), m),
                  pl.BlockSpec((blk, 1), m),
                  pl.BlockSpec((blk, 1), m)],
        out_specs=pl.BlockSpec((blk, 128), m),
        out_shape=jax.ShapeDtypeStruct((S * SUB, 128), jnp.float32),
        compiler_params=pltpu.CompilerParams(
            dimension_semantics=("arbitrary",)),
    )(shared_lin, r1_lin, r2_lin, w1s, w2s)


def kernel(hidden_states, norm_w, router_w, sh_fc1_w, sh_fc1_b, sh_fc2_w,
           sh_fc2_b, ex_fc1_w, ex_fc1_b, ex_fc2_w, ex_fc2_b):
    x = hidden_states.reshape(S, D)
    rwt = router_w.T
    sh1t = sh_fc1_w.T.astype(jnp.bfloat16)
    sh2t = sh_fc2_w.T.astype(jnp.bfloat16)
    w1t = ex_fc1_w.transpose(0, 2, 1).astype(jnp.bfloat16)
    w2t = ex_fc2_w.transpose(0, 2, 1).astype(jnp.bfloat16)

    ltri = jnp.tril(jnp.ones((S, S), jnp.bfloat16), -1)
    normed, shared, d1s, d2s, w1o, w2o, ex32 = _run_a1(
        x, norm_w, rwt, ltri, sh1t, sh_fc1_b, sh2t, sh_fc2_b)
    ex_tab = ex32.reshape(32)[:NSLOT]
    d1m = d1s.reshape(S * SUB // 128, 128)
    d2m = d2s.reshape(S * SUB // 128, 128)
    xp_lin = normed.reshape(S * SUB, 128)

    x_padded_lin = _run_b(xp_lin, d1m, d2m)
    y_padded_lin = _run_c(ex_tab, x_padded_lin, w1t, ex_fc1_b, w2t, ex_fc2_b)
    r1_lin, r2_lin = _run_d(y_padded_lin, d1m, d2m)

    shared_lin = shared.reshape(S * SUB, 128)
    w1s = jnp.repeat(w1o, SUB, axis=0)
    w2s = jnp.repeat(w2o, SUB, axis=0)
    out_lin = _run_e(shared_lin, r1_lin, r2_lin, w1s, w2s)
    return out_lin.reshape(B, S, O)


# speedup vs baseline: 1.4675x; 1.4675x over previous
import jax
import jax.numpy as jnp
from jax.experimental import pallas as pl
from jax.experimental.pallas import tpu as pltpu

B, S, D, H, O, E, K = 1, 2048, 1024, 1024, 1024, 8, 2
TB = 512


def _dot(a, b, precision=None):
    return jax.lax.dot_general(
        a, b, (((1,), (0,)), ((), ())),
        precision=precision, preferred_element_type=jnp.float32)


def _moe_body(x_ref, nw_ref, rwt_ref, sh1t_ref, sh1b_ref, sh2t_ref, sh2b_ref,
              w1t_ref, b1_ref, w2t_ref, b2_ref, o_ref):
    x = x_ref[...]
    var = jnp.mean(x * x, axis=-1, keepdims=True)
    normed = x * jax.lax.rsqrt(var + 1e-8) * nw_ref[...]

    logits = _dot(normed, rwt_ref[...])
    probs = 1.0 / (1.0 + jnp.exp(-logits))
    eidx = jax.lax.broadcasted_iota(jnp.int32, probs.shape, 1)
    m1 = jnp.max(probs, axis=-1, keepdims=True)
    i1 = jnp.min(jnp.where(probs == m1, eidx, E), axis=-1, keepdims=True)
    probs2 = jnp.where(eidx == i1, -1.0, probs)
    m2 = jnp.max(probs2, axis=-1, keepdims=True)
    i2 = jnp.min(jnp.where(probs2 == m2, eidx, E), axis=-1, keepdims=True)
    denom = m1 + m2 + 1e-6
    cw = (jnp.where(eidx == i1, m1, 0.0) + jnp.where(eidx == i2, m2, 0.0)) / denom

    nb = normed.astype(jnp.bfloat16)
    h = jnp.maximum(_dot(nb, sh1t_ref[...]) + sh1b_ref[...], 0.0)
    acc = _dot(h.astype(jnp.bfloat16), sh2t_ref[...]) + sh2b_ref[...]
    for e in range(E):
        he = jnp.maximum(_dot(nb, w1t_ref[e]) + b1_ref[e], 0.0)
        ye = _dot(he.astype(jnp.bfloat16), w2t_ref[e]) + b2_ref[e]
        acc = acc + cw[:, e:e + 1] * ye
    o_ref[...] = acc


def kernel(hidden_states, norm_w, router_w, sh_fc1_w, sh_fc1_b, sh_fc2_w,
           sh_fc2_b, ex_fc1_w, ex_fc1_b, ex_fc2_w, ex_fc2_b):
    x = hidden_states.reshape(S, D)
    rwt = router_w.T
    sh1t = sh_fc1_w.T.astype(jnp.bfloat16)
    sh2t = sh_fc2_w.T.astype(jnp.bfloat16)
    w1t = ex_fc1_w.transpose(0, 2, 1).astype(jnp.bfloat16)
    w2t = ex_fc2_w.transpose(0, 2, 1).astype(jnp.bfloat16)

    grid = (S // TB,)
    tok = lambda i: (i, 0)
    whole2 = lambda i: (0, 0)
    whole3 = lambda i: (0, 0, 0)
    out = pl.pallas_call(
        _moe_body,
        grid=grid,
        in_specs=[
            pl.BlockSpec((TB, D), tok),
            pl.BlockSpec((1, D), whole2),
            pl.BlockSpec((D, E), whole2),
            pl.BlockSpec((D, H), whole2),
            pl.BlockSpec((1, H), whole2),
            pl.BlockSpec((H, O), whole2),
            pl.BlockSpec((1, O), whole2),
            pl.BlockSpec((E, D, H), whole3),
            pl.BlockSpec((E, H), whole2),
            pl.BlockSpec((E, H, O), whole3),
            pl.BlockSpec((E, O), whole2),
        ],
        out_specs=pl.BlockSpec((TB, O), tok),
        out_shape=jax.ShapeDtypeStruct((S, O), jnp.float32),
        compiler_params=pltpu.CompilerParams(
            dimension_semantics=("arbitrary",),
        ),
    )(x, norm_w.reshape(1, D), rwt, sh1t, sh_fc1_b.reshape(1, H), sh2t,
      sh_fc2_b.reshape(1, O), w1t, ex_fc1_b, w2t, ex_fc2_b)
    return out.reshape(B, S, O)
